# Initial kernel scaffold; baseline (speedup 1.0000x reference)
import jax
import jax.numpy as jnp
from jax.experimental import pallas as pl


def kernel(src, dst, edge_type, head_ids, tail_ids, rel_labels, rel_vectors, W1, b1, W2, b2, reld_W, reld_b, conc_W, conc_b, fc_W, fc_b):
    raise NotImplementedError("write your pallas kernel here")


# speedup vs baseline: 7.7505x; 7.7505x over previous
import jax
import jax.numpy as jnp
from jax.experimental import pallas as pl
from jax.experimental.pallas import tpu as pltpu
import functools


def _hist_kernel(src_ref, dst_ref, et_ref, head_ref, tail_ref, lab_ref,
                 rv_ref, W1_ref, b1_ref, W2_ref, b2_ref,
                 reldW_ref, reldb_ref, concW_ref, concb_ref, fcW_ref, fcb_ref,
                 out_ref, acc_ref, *, nb, n_rels, b_rows, link_mode):
    i = pl.program_id(0)
    s = src_ref[0, 0, :]
    d = dst_ref[0, 0, :]
    t = et_ref[0, 0, :]
    h = head_ref[:, 0]
    tl = tail_ref[:, 0]

    hs = s[None, :] == h[:, None]
    hd = d[None, :] == h[:, None]
    ts = s[None, :] == tl[:, None]
    td = d[None, :] == tl[:, None]
    m5 = hs & td
    m6 = hd & ts

    rows = jnp.concatenate([
        hd & ~m6,
        hs & ~m5,
        td & ~m5,
        ts & ~m6,
        m5,
        m6,
    ], axis=0).astype(jnp.bfloat16)

    k = t.shape[0]
    oh = (t[:, None] == jax.lax.broadcasted_iota(jnp.int32, (k, n_rels), 1)
          ).astype(jnp.bfloat16)

    c_blk = jnp.dot(rows, oh, preferred_element_type=jnp.float32)

    @pl.when(i == 0)
    def _init():
        acc_ref[...] = c_blk

    @pl.when(i > 0)
    def _accum():
        acc_ref[...] = acc_ref[...] + c_blk

    @pl.when(i == nb - 1)
    def _tail():
        C = acc_ref[...]
        rv = rv_ref[...]
        V = (jnp.dot(rv, W1_ref[...], preferred_element_type=jnp.float32)
             + b1_ref[...])
        V = (jnp.dot(V, W2_ref[...], preferred_element_type=jnp.float32)
             + b2_ref[...])
        S = jnp.dot(C, V, preferred_element_type=jnp.float32)
        n = jnp.sum(C, axis=1, keepdims=True)
        acc = jnp.zeros((b_rows, V.shape[1]), jnp.float32)
        for m in range(link_mode):
            Sm = S[m * b_rows:(m + 1) * b_rows, :]
            nm = n[m * b_rows:(m + 1) * b_rows, :]
            Tm = (jnp.dot(Sm, reldW_ref[m], preferred_element_type=jnp.float32)
                  + nm * reldb_ref[m, :][None, :])
            acc = acc + Tm / (nm + 1e-30)
        rel_neighbor = acc / float(link_mode)

        lab = lab_ref[:, 0]
        loh = (lab[:, None] == jax.lax.broadcasted_iota(
            jnp.int32, (b_rows, n_rels), 1)).astype(jnp.float32)
        rel_embeds = jnp.dot(loh, V, preferred_element_type=jnp.float32)

        hcat = jnp.concatenate([rel_neighbor, rel_embeds], axis=1)
        hh = (jnp.dot(hcat, concW_ref[...], preferred_element_type=jnp.float32)
              + concb_ref[...])
        hh = jnp.maximum(hh, 0.0)
        nrm = jnp.sqrt(jnp.sum(hh * hh, axis=1, keepdims=True))
        g = hh / jnp.maximum(nrm, 1e-12)
        out_ref[...] = (jnp.dot(g, fcW_ref[...], preferred_element_type=jnp.float32)
                        + fcb_ref[...])


def kernel(src, dst, edge_type, head_ids, tail_ids, rel_labels, rel_vectors,
           W1, b1, W2, b2, reld_W, reld_b, conc_W, conc_b, fc_W, fc_b):
    E = src.shape[0]
    B = head_ids.shape[0]
    R, DV = rel_vectors.shape
    D = W1.shape[1]
    L = reld_W.shape[0]

    K = 4000
    assert E % K == 0
    NB = E // K

    src3 = src.reshape(NB, 1, K)
    dst3 = dst.reshape(NB, 1, K)
    et3 = edge_type.reshape(NB, 1, K)

    def blk(shape):
        return pl.BlockSpec(shape, lambda i, _s=shape: tuple(0 for _ in _s))

    edge_spec = pl.BlockSpec((1, 1, K), lambda i: (i, 0, 0))

    out = pl.pallas_call(
        functools.partial(_hist_kernel, nb=NB, n_rels=R, b_rows=B,
                          link_mode=L),
        grid=(NB,),
        in_specs=[
            edge_spec, edge_spec, edge_spec,
            blk((B, 1)), blk((B, 1)), blk((B, 1)),
            blk((R, DV)),
            blk((DV, D)), blk((1, D)),
            blk((D, D)), blk((1, D)),
            blk((L, D, D)), blk((L, D)),
            blk((2 * D, D)), blk((1, D)),
            blk((D, 1)), blk((1, 1)),
        ],
        out_specs=blk((B, 1)),
        out_shape=jax.ShapeDtypeStruct((B, 1), jnp.float32),
        scratch_shapes=[pltpu.VMEM((L * B, R), jnp.float32)],
    )(src3, dst3, et3,
      head_ids.reshape(B, 1), tail_ids.reshape(B, 1), rel_labels.reshape(B, 1),
      rel_vectors, W1, b1.reshape(1, D), W2, b2.reshape(1, D),
      reld_W, reld_b, conc_W, conc_b.reshape(1, D),
      fc_W, fc_b.reshape(1, 1))
    return out


# speedup vs baseline: 7.8002x; 1.0064x over previous
import functools
import numpy as np
import jax
import jax.numpy as jnp
from jax import lax
from jax.experimental import pallas as pl
from jax.experimental.pallas import tpu as pltpu
from jax.experimental.pallas import tpu_sc as plsc

E_EDGES = 160000
NWORKERS = 32
CHUNK = E_EDGES // NWORKERS
CPAD = CHUNK + 8
NV = CPAD // 16
EBUF = CPAD + 16
N_NODES = 10000
NPAD = N_NODES + 32
PADNODE = N_NODES
B = 32
R = 200
L = 6
CSIZE = L * B * R

_BITS = [int(np.int32(np.uint32(1 << b))) for b in range(32)]


def _sc_hist(src_hbm, dst_hbm, et_hbm, head_hbm, tail_hbm, out_hbm,
             sv, dv, tv, htbl, ttbl, queue, cbuf, hv, tlv):
    wid = lax.axis_index("c") * 16 + lax.axis_index("s")
    base = wid * CHUNK
    pltpu.sync_copy(src_hbm.at[pl.ds(base, CHUNK)], sv.at[pl.ds(0, CHUNK)])
    pltpu.sync_copy(dst_hbm.at[pl.ds(base, CHUNK)], dv.at[pl.ds(0, CHUNK)])
    pltpu.sync_copy(et_hbm.at[pl.ds(base, CHUNK)], tv.at[pl.ds(0, CHUNK)])
    pltpu.sync_copy(head_hbm, hv)
    pltpu.sync_copy(tail_hbm, tlv)

    lanes = lax.iota(jnp.int32, 16)
    vmask = lanes < 8
    sv[pl.ds(CHUNK - 8, 16)] = jnp.where(vmask, sv[pl.ds(CHUNK - 8, 16)],
                                         PADNODE)
    dv[pl.ds(CHUNK - 8, 16)] = jnp.where(vmask, dv[pl.ds(CHUNK - 8, 16)],
                                         PADNODE)

    zi = jnp.zeros((16,), jnp.int32)

    def _ztbl(j, c):
        htbl[pl.ds(j * 16, 16)] = zi
        ttbl[pl.ds(j * 16, 16)] = zi
        return c
    lax.fori_loop(0, NPAD // 16, _ztbl, 0)

    zf = jnp.zeros((16,), jnp.float32)

    def _zc(j, c):
        cbuf[pl.ds(j * 16, 16)] = zf
        return c
    lax.fori_loop(0, CSIZE // 16, _zc, 0)

    lane0 = lanes == 0
    hv0 = hv[pl.ds(0, 16)]
    hv1 = hv[pl.ds(16, 16)]
    tv0 = tlv[pl.ds(0, 16)]
    tv1 = tlv[pl.ds(16, 16)]
    for b in range(B):
        hid = hv0[b] if b < 16 else hv1[b - 16]
        tid = tv0[b] if b < 16 else tv1[b - 16]
        bit = jnp.full((16,), _BITS[b], jnp.int32)
        hidx = jnp.full((16,), hid, jnp.int32)
        tidx = jnp.full((16,), tid, jnp.int32)
        hold = plsc.load_gather(htbl, [hidx], mask=lane0)
        plsc.store_scatter(htbl, [hidx], hold | bit, mask=lane0)
        told = plsc.load_gather(ttbl, [tidx], mask=lane0)
        plsc.store_scatter(ttbl, [tidx], told | bit, mask=lane0)

    def _scan(j, cnt):
        off = j * 16
        s16 = sv[pl.ds(off, 16)]
        d16 = dv[pl.ds(off, 16)]
        hs = plsc.load_gather(htbl, [s16])
        hd = plsc.load_gather(htbl, [d16])
        ts = plsc.load_gather(ttbl, [s16])
        td = plsc.load_gather(ttbl, [d16])
        anyv = (hs | hd) | (ts | td)
        msk = anyv != 0
        plsc.store_compressed(queue.at[pl.ds(cnt, 16)], off + lanes, mask=msk)
        return cnt + jnp.sum(msk.astype(jnp.int32))

    cnt = lax.fori_loop(0, NV, _scan, jnp.int32(0))

    onesf = jnp.ones((16,), jnp.float32)

    def _proc(k, c):
        e = queue[pl.ds(k, 16)][0]
        s = sv[pl.ds(e, 16)][0]
        d = dv[pl.ds(e, 16)][0]
        t = tv[pl.ds(e, 16)][0]
        hs = htbl[pl.ds(s, 16)][0]
        hd = htbl[pl.ds(d, 16)][0]
        ts = ttbl[pl.ds(s, 16)][0]
        td = ttbl[pl.ds(d, 16)][0]
        m5 = hs & td
        m6 = hd & ts
        modes = [hd & ~m6, hs & ~m5, td & ~m5, ts & ~m6, m5, m6]
        for half in range(2):
            bidx = lanes + (16 * half)
            for i in range(L):
                bits = jnp.right_shift(modes[i], bidx) & 1
                msk = bits == 1
                idx = (i * (B * R) + t) + bidx * R
                old = plsc.load_gather(cbuf, [idx], mask=msk)
                plsc.store_scatter(cbuf, [idx], old + onesf, mask=msk)
        return c

    lax.fori_loop(0, cnt, _proc, 0)
    pltpu.sync_copy(cbuf, out_hbm.at[wid])


def _tail_kernel(cp_ref, lab_ref, rv_ref, W1_ref, b1_ref, W2_ref, b2_ref,
                 reldW_ref, reldb_ref, concW_ref, concb_ref, fcW_ref, fcb_ref,
                 out_ref, *, n_rels, b_rows, link_mode):
    C = jnp.sum(cp_ref[...], axis=0)
    V = (jnp.dot(rv_ref[...], W1_ref[...],
                 preferred_element_type=jnp.float32) + b1_ref[...])
    V = (jnp.dot(V, W2_ref[...],
                 preferred_element_type=jnp.float32) + b2_ref[...])
    S = jnp.dot(C, V, preferred_element_type=jnp.float32)
    n = jnp.sum(C, axis=1, keepdims=True)
    acc = jnp.zeros((b_rows, V.shape[1]), jnp.float32)
    for m in range(link_mode):
        Sm = S[m * b_rows:(m + 1) * b_rows, :]
        nm = n[m * b_rows:(m + 1) * b_rows, :]
        Tm = (jnp.dot(Sm, reldW_ref[m], preferred_element_type=jnp.float32)
              + nm * reldb_ref[m, :][None, :])
        acc = acc + Tm / (nm + 1e-30)
    rel_neighbor = acc / float(link_mode)

    lab = lab_ref[:, 0]
    loh = (lab[:, None] == jax.lax.broadcasted_iota(
        jnp.int32, (b_rows, n_rels), 1)).astype(jnp.float32)
    rel_embeds = jnp.dot(loh, V, preferred_element_type=jnp.float32)

    hcat = jnp.concatenate([rel_neighbor, rel_embeds], axis=1)
    hh = (jnp.dot(hcat, concW_ref[...], preferred_element_type=jnp.float32)
          + concb_ref[...])
    hh = jnp.maximum(hh, 0.0)
    nrm = jnp.sqrt(jnp.sum(hh * hh, axis=1, keepdims=True))
    g = hh / jnp.maximum(nrm, 1e-12)
    out_ref[...] = (jnp.dot(g, fcW_ref[...],
                            preferred_element_type=jnp.float32) + fcb_ref[...])


def kernel(src, dst, edge_type, head_ids, tail_ids, rel_labels, rel_vectors,
           W1, b1, W2, b2, reld_W, reld_b, conc_W, conc_b, fc_W, fc_b):
    DV = rel_vectors.shape[1]
    D = W1.shape[1]

    mesh = plsc.VectorSubcoreMesh(core_axis_name="c", subcore_axis_name="s")

    hist = pl.kernel(
        _sc_hist,
        mesh=mesh,
        compiler_params=pltpu.CompilerParams(needs_layout_passes=False),
        out_type=jax.ShapeDtypeStruct((NWORKERS, CSIZE), jnp.float32),
        scratch_types=[
            pltpu.VMEM((EBUF,), jnp.int32),
            pltpu.VMEM((EBUF,), jnp.int32),
            pltpu.VMEM((EBUF,), jnp.int32),
            pltpu.VMEM((NPAD,), jnp.int32),
            pltpu.VMEM((NPAD,), jnp.int32),
            pltpu.VMEM((EBUF,), jnp.int32),
            pltpu.VMEM((CSIZE,), jnp.float32),
            pltpu.VMEM((B,), jnp.int32),
            pltpu.VMEM((B,), jnp.int32),
        ],
    )
    cparts = hist(src, dst, edge_type, head_ids, tail_ids)
    cparts = cparts.reshape(NWORKERS, L * B, R)

    def blk(shape):
        return pl.BlockSpec(shape, lambda *, _s=shape: tuple(0 for _ in _s))

    out = pl.pallas_call(
        functools.partial(_tail_kernel, n_rels=R, b_rows=B, link_mode=L),
        in_specs=[
            blk((NWORKERS, L * B, R)),
            blk((B, 1)),
            blk((R, DV)),
            blk((DV, D)), blk((1, D)),
            blk((D, D)), blk((1, D)),
            blk((L, D, D)), blk((L, D)),
            blk((2 * D, D)), blk((1, D)),
            blk((D, 1)), blk((1, 1)),
        ],
        out_specs=blk((B, 1)),
        out_shape=jax.ShapeDtypeStruct((B, 1), jnp.float32),
    )(cparts, rel_labels.reshape(B, 1),
      rel_vectors, W1, b1.reshape(1, D), W2, b2.reshape(1, D),
      reld_W, reld_b, conc_W, conc_b.reshape(1, D),
      fc_W, fc_b.reshape(1, 1))
    return out


# speedup vs baseline: 11.0845x; 1.4211x over previous
import functools
import numpy as np
import jax
import jax.numpy as jnp
from jax import lax
from jax.experimental import pallas as pl
from jax.experimental.pallas import tpu as pltpu
from jax.experimental.pallas import tpu_sc as plsc

E_EDGES = 160000
NWORKERS = 32
CHUNK = E_EDGES // NWORKERS
CPAD = CHUNK + 8
NV = CPAD // 16
EBUF = CPAD + 16
N_NODES = 10000
NPAD = N_NODES + 32
PADNODE = N_NODES
B = 32
R = 200
L = 6
LB = L * B

_BITS = [int(np.int32(np.uint32(1 << b))) for b in range(32)]


def _sc_hist(src_hbm, dst_hbm, et_hbm, head_hbm, tail_hbm, zt_hbm, zc_hbm,
             out_hbm, sv, dv, tv, htbl, ttbl, atbl, queue, cbuf, hv, tlv,
             sem, csem):
    wid = lax.axis_index("c") * 16 + lax.axis_index("s")
    base = wid * CHUNK
    cz = pltpu.async_copy(zc_hbm, cbuf, csem)
    z1 = pltpu.async_copy(zt_hbm, htbl, sem)
    z2 = pltpu.async_copy(zt_hbm, ttbl, sem)
    z3 = pltpu.async_copy(zt_hbm, atbl, sem)
    c1 = pltpu.async_copy(src_hbm.at[pl.ds(base, CHUNK)],
                          sv.at[pl.ds(0, CHUNK)], sem)
    c2 = pltpu.async_copy(dst_hbm.at[pl.ds(base, CHUNK)],
                          dv.at[pl.ds(0, CHUNK)], sem)
    c3 = pltpu.async_copy(et_hbm.at[pl.ds(base, CHUNK)],
                          tv.at[pl.ds(0, CHUNK)], sem)
    c4 = pltpu.async_copy(head_hbm, hv, sem)
    c5 = pltpu.async_copy(tail_hbm, tlv, sem)
    for c in (z1, z2, z3, c1, c2, c3, c4, c5):
        c.wait()

    lanes = lax.iota(jnp.int32, 16)
    vmask = lanes < 8
    sv[pl.ds(CHUNK - 8, 16)] = jnp.where(vmask, sv[pl.ds(CHUNK - 8, 16)],
                                         PADNODE)
    dv[pl.ds(CHUNK - 8, 16)] = jnp.where(vmask, dv[pl.ds(CHUNK - 8, 16)],
                                         PADNODE)

    lane0 = lanes == 0
    hv0 = hv[pl.ds(0, 16)]
    hv1 = hv[pl.ds(16, 16)]
    tv0 = tlv[pl.ds(0, 16)]
    tv1 = tlv[pl.ds(16, 16)]
    for b in range(B):
        hid = hv0[b] if b < 16 else hv1[b - 16]
        tid = tv0[b] if b < 16 else tv1[b - 16]
        bit = jnp.full((16,), _BITS[b], jnp.int32)
        hidx = jnp.full((16,), hid, jnp.int32)
        tidx = jnp.full((16,), tid, jnp.int32)
        hold = plsc.load_gather(htbl, [hidx], mask=lane0)
        plsc.store_scatter(htbl, [hidx], hold | bit, mask=lane0)
        told = plsc.load_gather(ttbl, [tidx], mask=lane0)
        plsc.store_scatter(ttbl, [tidx], told | bit, mask=lane0)
        ahold = plsc.load_gather(atbl, [hidx], mask=lane0)
        plsc.store_scatter(atbl, [hidx], ahold | bit, mask=lane0)
        atold = plsc.load_gather(atbl, [tidx], mask=lane0)
        plsc.store_scatter(atbl, [tidx], atold | bit, mask=lane0)

    def _scan(j, cnt):
        off = j * 16
        s16 = sv[pl.ds(off, 16)]
        d16 = dv[pl.ds(off, 16)]
        a_s = plsc.load_gather(atbl, [s16])
        a_d = plsc.load_gather(atbl, [d16])
        msk = (a_s | a_d) != 0
        plsc.store_compressed(queue.at[pl.ds(cnt, 16)], off + lanes, mask=msk)
        return cnt + plsc.all_reduce_population_count(msk)[0]

    cnt = lax.fori_loop(0, NV, _scan, jnp.int32(0))

    cz.wait()
    onesf = jnp.ones((16,), jnp.float32)

    def _proc(k, c):
        e = queue[pl.ds(k, 16)][0]
        s = sv[pl.ds(e, 16)][0]
        d = dv[pl.ds(e, 16)][0]
        t = tv[pl.ds(e, 16)][0]
        hs = htbl[pl.ds(s, 16)][0]
        hd = htbl[pl.ds(d, 16)][0]
        ts = ttbl[pl.ds(s, 16)][0]
        td = ttbl[pl.ds(d, 16)][0]
        m5 = hs & td
        m6 = hd & ts
        modes = [hd & ~m6, hs & ~m5, td & ~m5, ts & ~m6, m5, m6]
        tvec = jnp.full((16,), t, jnp.int32)
        for half in range(2):
            bidx = lanes + (16 * half)
            for i in range(L):
                bits = jnp.right_shift(modes[i], bidx) & 1
                msk = bits == 1
                col = bidx + (i * B)
                plsc.addupdate_scatter(cbuf, [tvec, col], onesf, mask=msk)
        return c

    lax.fori_loop(0, cnt, _proc, 0)
    pltpu.sync_copy(cbuf, out_hbm.at[wid])


def _tail_kernel(cp_ref, lab_ref, rv_ref, W1_ref, b1_ref, W2_ref, b2_ref,
                 reldW_ref, reldb_ref, concW_ref, concb_ref, fcW_ref, fcb_ref,
                 out_ref, *, n_rels, b_rows, link_mode):
    CT = jnp.sum(cp_ref[...], axis=0)
    V = (jnp.dot(rv_ref[...], W1_ref[...],
                 preferred_element_type=jnp.float32) + b1_ref[...])
    V = (jnp.dot(V, W2_ref[...],
                 preferred_element_type=jnp.float32) + b2_ref[...])
    S = lax.dot_general(CT, V, (((0,), (0,)), ((), ())),
                        preferred_element_type=jnp.float32)
    n = jnp.sum(CT, axis=0)[:, None]
    acc = jnp.zeros((b_rows, V.shape[1]), jnp.float32)
    for m in range(link_mode):
        Sm = S[m * b_rows:(m + 1) * b_rows, :]
        nm = n[m * b_rows:(m + 1) * b_rows, :]
        Tm = (jnp.dot(Sm, reldW_ref[m], preferred_element_type=jnp.float32)
              + nm * reldb_ref[m, :][None, :])
        acc = acc + Tm / (nm + 1e-30)
    rel_neighbor = acc / float(link_mode)

    lab = lab_ref[:, 0]
    loh = (lab[:, None] == jax.lax.broadcasted_iota(
        jnp.int32, (b_rows, n_rels), 1)).astype(jnp.float32)
    rel_embeds = jnp.dot(loh, V, preferred_element_type=jnp.float32)

    hcat = jnp.concatenate([rel_neighbor, rel_embeds], axis=1)
    hh = (jnp.dot(hcat, concW_ref[...], preferred_element_type=jnp.float32)
          + concb_ref[...])
    hh = jnp.maximum(hh, 0.0)
    nrm = jnp.sqrt(jnp.sum(hh * hh, axis=1, keepdims=True))
    g = hh / jnp.maximum(nrm, 1e-12)
    out_ref[...] = (jnp.dot(g, fcW_ref[...],
                            preferred_element_type=jnp.float32) + fcb_ref[...])


def kernel(src, dst, edge_type, head_ids, tail_ids, rel_labels, rel_vectors,
           W1, b1, W2, b2, reld_W, reld_b, conc_W, conc_b, fc_W, fc_b):
    DV = rel_vectors.shape[1]
    D = W1.shape[1]

    mesh = plsc.VectorSubcoreMesh(core_axis_name="c", subcore_axis_name="s")

    hist = pl.kernel(
        _sc_hist,
        mesh=mesh,
        compiler_params=pltpu.CompilerParams(needs_layout_passes=False),
        out_type=jax.ShapeDtypeStruct((NWORKERS, R, LB), jnp.float32),
        scratch_types=[
            pltpu.VMEM((EBUF,), jnp.int32),
            pltpu.VMEM((EBUF,), jnp.int32),
            pltpu.VMEM((EBUF,), jnp.int32),
            pltpu.VMEM((NPAD,), jnp.int32),
            pltpu.VMEM((NPAD,), jnp.int32),
            pltpu.VMEM((NPAD,), jnp.int32),
            pltpu.VMEM((EBUF,), jnp.int32),
            pltpu.VMEM((R, LB), jnp.float32),
            pltpu.VMEM((B,), jnp.int32),
            pltpu.VMEM((B,), jnp.int32),
            pltpu.SemaphoreType.DMA,
            pltpu.SemaphoreType.DMA,
        ],
    )
    zt = jnp.zeros((NPAD,), jnp.int32)
    zc = jnp.zeros((R, LB), jnp.float32)
    cparts = hist(src, dst, edge_type, head_ids, tail_ids, zt, zc)

    def blk(shape):
        return pl.BlockSpec(shape, lambda *, _s=shape: tuple(0 for _ in _s))

    out = pl.pallas_call(
        functools.partial(_tail_kernel, n_rels=R, b_rows=B, link_mode=L),
        in_specs=[
            blk((NWORKERS, R, LB)),
            blk((B, 1)),
            blk((R, DV)),
            blk((DV, D)), blk((1, D)),
            blk((D, D)), blk((1, D)),
            blk((L, D, D)), blk((L, D)),
            blk((2 * D, D)), blk((1, D)),
            blk((D, 1)), blk((1, 1)),
        ],
        out_specs=blk((B, 1)),
        out_shape=jax.ShapeDtypeStruct((B, 1), jnp.float32),
    )(cparts, rel_labels.reshape(B, 1),
      rel_vectors, W1, b1.reshape(1, D), W2, b2.reshape(1, D),
      reld_W, reld_b, conc_W, conc_b.reshape(1, D),
      fc_W, fc_b.reshape(1, 1))
    return out


# speedup vs baseline: 11.4487x; 1.0329x over previous
import functools
import numpy as np
import jax
import jax.numpy as jnp
from jax import lax
from jax.experimental import pallas as pl
from jax.experimental.pallas import tpu as pltpu
from jax.experimental.pallas import tpu_sc as plsc

E_EDGES = 160000
NWORKERS = 32
CHUNK = E_EDGES // NWORKERS
CPAD = CHUNK + 56
NV = CPAD // 16
EBUF = CPAD + 32
N_NODES = 10000
NPAD = N_NODES + 32
PADNODE = N_NODES
B = 32
R = 200
L = 6
LB = L * B

_BITS = [int(np.int32(np.uint32(1 << b))) for b in range(32)]


def _sc_hist(src_hbm, dst_hbm, et_hbm, head_hbm, tail_hbm, zt_hbm, zc_hbm,
             out_hbm, sv, dv, tv, htbl, ttbl, atbl, queue, cbuf, hv, tlv,
             sem, csem):
    wid = lax.axis_index("c") * 16 + lax.axis_index("s")
    base = wid * CHUNK
    cz = pltpu.async_copy(zc_hbm, cbuf, csem)
    z1 = pltpu.async_copy(zt_hbm, htbl, sem)
    z2 = pltpu.async_copy(zt_hbm, ttbl, sem)
    z3 = pltpu.async_copy(zt_hbm, atbl, sem)
    c1 = pltpu.async_copy(src_hbm.at[pl.ds(base, CHUNK)],
                          sv.at[pl.ds(0, CHUNK)], sem)
    c2 = pltpu.async_copy(dst_hbm.at[pl.ds(base, CHUNK)],
                          dv.at[pl.ds(0, CHUNK)], sem)
    c3 = pltpu.async_copy(et_hbm.at[pl.ds(base, CHUNK)],
                          tv.at[pl.ds(0, CHUNK)], sem)
    c4 = pltpu.async_copy(head_hbm, hv, sem)
    c5 = pltpu.async_copy(tail_hbm, tlv, sem)
    for c in (z1, z2, z3, c1, c2, c3, c4, c5):
        c.wait()

    lanes = lax.iota(jnp.int32, 16)
    vmask = lanes < 8
    padv = jnp.full((16,), PADNODE, jnp.int32)
    sv[pl.ds(CHUNK - 8, 16)] = jnp.where(vmask, sv[pl.ds(CHUNK - 8, 16)],
                                         PADNODE)
    dv[pl.ds(CHUNK - 8, 16)] = jnp.where(vmask, dv[pl.ds(CHUNK - 8, 16)],
                                         PADNODE)
    for p in range(CHUNK + 8, CPAD, 16):
        sv[pl.ds(p, 16)] = padv
        dv[pl.ds(p, 16)] = padv

    lane0 = lanes == 0
    hv0 = hv[pl.ds(0, 16)]
    hv1 = hv[pl.ds(16, 16)]
    tv0 = tlv[pl.ds(0, 16)]
    tv1 = tlv[pl.ds(16, 16)]
    for b in range(B):
        hid = hv0[b] if b < 16 else hv1[b - 16]
        tid = tv0[b] if b < 16 else tv1[b - 16]
        bit = jnp.full((16,), _BITS[b], jnp.int32)
        hidx = jnp.full((16,), hid, jnp.int32)
        tidx = jnp.full((16,), tid, jnp.int32)
        hold = plsc.load_gather(htbl, [hidx], mask=lane0)
        plsc.store_scatter(htbl, [hidx], hold | bit, mask=lane0)
        told = plsc.load_gather(ttbl, [tidx], mask=lane0)
        plsc.store_scatter(ttbl, [tidx], told | bit, mask=lane0)
        ahold = plsc.load_gather(atbl, [hidx], mask=lane0)
        plsc.store_scatter(atbl, [hidx], ahold | bit, mask=lane0)
        atold = plsc.load_gather(atbl, [tidx], mask=lane0)
        plsc.store_scatter(atbl, [tidx], atold | bit, mask=lane0)

    def _scan(j, cnt):
        base4 = j * 64
        ss = [sv[pl.ds(base4 + u * 16, 16)] for u in range(4)]
        dd = [dv[pl.ds(base4 + u * 16, 16)] for u in range(4)]
        aas = [plsc.load_gather(atbl, [s]) for s in ss]
        aad = [plsc.load_gather(atbl, [d]) for d in dd]
        msks = [(aas[u] | aad[u]) != 0 for u in range(4)]
        for u in range(4):
            plsc.store_compressed(queue.at[pl.ds(cnt, 16)],
                                  base4 + u * 16 + lanes, mask=msks[u])
            cnt = cnt + plsc.all_reduce_population_count(msks[u])[0]
        return cnt

    cnt = lax.fori_loop(0, NV // 4, _scan, jnp.int32(0))

    cz.wait()
    onesf = jnp.ones((16,), jnp.float32)
    queue[pl.ds(cnt, 16)] = jnp.full((16,), CHUNK, jnp.int32)

    def _proc(k, c):
        e16 = queue[pl.ds(k * 16, 16)]
        s16 = plsc.load_gather(sv, [e16])
        d16 = plsc.load_gather(dv, [e16])
        t16 = plsc.load_gather(tv, [e16])
        hs = plsc.load_gather(htbl, [s16])
        hd = plsc.load_gather(htbl, [d16])
        ts = plsc.load_gather(ttbl, [s16])
        td = plsc.load_gather(ttbl, [d16])
        m5 = hs & td
        m6 = hd & ts
        modes = [hd & ~m6, hs & ~m5, td & ~m5, ts & ~m6, m5, m6]
        for j in range(16):
            tvec = jnp.full((16,), t16[j], jnp.int32)
            mj = [jnp.full((16,), modes[i][j], jnp.int32) for i in range(L)]
            for half in range(2):
                bidx = lanes + (16 * half)
                for i in range(L):
                    bits = jnp.right_shift(mj[i], bidx) & 1
                    msk = bits == 1
                    col = bidx + (i * B)
                    plsc.addupdate_scatter(cbuf, [tvec, col], onesf, mask=msk)
        return c

    lax.fori_loop(0, (cnt + 15) // 16, _proc, 0)
    pltpu.sync_copy(cbuf, out_hbm.at[wid])


def _tail_kernel(cp_ref, lab_ref, rv_ref, W1_ref, b1_ref, W2_ref, b2_ref,
                 reldW_ref, reldb_ref, concW_ref, concb_ref, fcW_ref, fcb_ref,
                 out_ref, *, n_rels, b_rows, link_mode):
    CT = jnp.sum(cp_ref[...], axis=0)
    V = (jnp.dot(rv_ref[...], W1_ref[...],
                 preferred_element_type=jnp.float32) + b1_ref[...])
    V = (jnp.dot(V, W2_ref[...],
                 preferred_element_type=jnp.float32) + b2_ref[...])
    S = lax.dot_general(CT, V, (((0,), (0,)), ((), ())),
                        preferred_element_type=jnp.float32)
    n = jnp.sum(CT, axis=0)[:, None]
    acc = jnp.zeros((b_rows, V.shape[1]), jnp.float32)
    for m in range(link_mode):
        Sm = S[m * b_rows:(m + 1) * b_rows, :]
        nm = n[m * b_rows:(m + 1) * b_rows, :]
        Tm = (jnp.dot(Sm, reldW_ref[m], preferred_element_type=jnp.float32)
              + nm * reldb_ref[m, :][None, :])
        acc = acc + Tm / (nm + 1e-30)
    rel_neighbor = acc / float(link_mode)

    lab = lab_ref[:, 0]
    loh = (lab[:, None] == jax.lax.broadcasted_iota(
        jnp.int32, (b_rows, n_rels), 1)).astype(jnp.float32)
    rel_embeds = jnp.dot(loh, V, preferred_element_type=jnp.float32)

    hcat = jnp.concatenate([rel_neighbor, rel_embeds], axis=1)
    hh = (jnp.dot(hcat, concW_ref[...], preferred_element_type=jnp.float32)
          + concb_ref[...])
    hh = jnp.maximum(hh, 0.0)
    nrm = jnp.sqrt(jnp.sum(hh * hh, axis=1, keepdims=True))
    g = hh / jnp.maximum(nrm, 1e-12)
    out_ref[...] = (jnp.dot(g, fcW_ref[...],
                            preferred_element_type=jnp.float32) + fcb_ref[...])


def kernel(src, dst, edge_type, head_ids, tail_ids, rel_labels, rel_vectors,
           W1, b1, W2, b2, reld_W, reld_b, conc_W, conc_b, fc_W, fc_b):
    DV = rel_vectors.shape[1]
    D = W1.shape[1]

    mesh = plsc.VectorSubcoreMesh(core_axis_name="c", subcore_axis_name="s")

    hist = pl.kernel(
        _sc_hist,
        mesh=mesh,
        compiler_params=pltpu.CompilerParams(needs_layout_passes=False),
        out_type=jax.ShapeDtypeStruct((NWORKERS, R, LB), jnp.float32),
        scratch_types=[
            pltpu.VMEM((EBUF,), jnp.int32),
            pltpu.VMEM((EBUF,), jnp.int32),
            pltpu.VMEM((EBUF,), jnp.int32),
            pltpu.VMEM((NPAD,), jnp.int32),
            pltpu.VMEM((NPAD,), jnp.int32),
            pltpu.VMEM((NPAD,), jnp.int32),
            pltpu.VMEM((EBUF,), jnp.int32),
            pltpu.VMEM((R, LB), jnp.float32),
            pltpu.VMEM((B,), jnp.int32),
            pltpu.VMEM((B,), jnp.int32),
            pltpu.SemaphoreType.DMA,
            pltpu.SemaphoreType.DMA,
        ],
    )
    zt = jnp.zeros((NPAD,), jnp.int32)
    zc = jnp.zeros((R, LB), jnp.float32)
    cparts = hist(src, dst, edge_type, head_ids, tail_ids, zt, zc)

    def blk(shape):
        return pl.BlockSpec(shape, lambda *, _s=shape: tuple(0 for _ in _s))

    out = pl.pallas_call(
        functools.partial(_tail_kernel, n_rels=R, b_rows=B, link_mode=L),
        in_specs=[
            blk((NWORKERS, R, LB)),
            blk((B, 1)),
            blk((R, DV)),
            blk((DV, D)), blk((1, D)),
            blk((D, D)), blk((1, D)),
            blk((L, D, D)), blk((L, D)),
            blk((2 * D, D)), blk((1, D)),
            blk((D, 1)), blk((1, 1)),
        ],
        out_specs=blk((B, 1)),
        out_shape=jax.ShapeDtypeStruct((B, 1), jnp.float32),
    )(cparts, rel_labels.reshape(B, 1),
      rel_vectors, W1, b1.reshape(1, D), W2, b2.reshape(1, D),
      reld_W, reld_b, conc_W, conc_b.reshape(1, D),
      fc_W, fc_b.reshape(1, 1))
    return out


# speedup vs baseline: 11.5134x; 1.0057x over previous
import functools
import numpy as np
import jax
import jax.numpy as jnp
from jax import lax
from jax.experimental import pallas as pl
from jax.experimental.pallas import tpu as pltpu
from jax.experimental.pallas import tpu_sc as plsc

E_EDGES = 160000
NWORKERS = 32
CHUNK = E_EDGES // NWORKERS
CPAD = CHUNK + 56
NV = CPAD // 16
EBUF = CPAD + 32
N_NODES = 10000
NPAD = N_NODES + 32
PADNODE = N_NODES
B = 32
R = 200
L = 6
LB = L * B

_BITS = [int(np.int32(np.uint32(1 << b))) for b in range(32)]


def _sc_hist(src_hbm, dst_hbm, et_hbm, head_hbm, tail_hbm, zt_hbm, zc_hbm,
             out_hbm, sv, dv, tv, htbl, ttbl, atbl, queue, cbuf, hv, tlv,
             sem, csem):
    wid = lax.axis_index("c") * 16 + lax.axis_index("s")
    base = wid * CHUNK
    cz = pltpu.async_copy(zc_hbm, cbuf, csem)
    z1 = pltpu.async_copy(zt_hbm, htbl, sem)
    z2 = pltpu.async_copy(zt_hbm, ttbl, sem)
    z3 = pltpu.async_copy(zt_hbm, atbl, sem)
    c1 = pltpu.async_copy(src_hbm.at[pl.ds(base, CHUNK)],
                          sv.at[pl.ds(0, CHUNK)], sem)
    c2 = pltpu.async_copy(dst_hbm.at[pl.ds(base, CHUNK)],
                          dv.at[pl.ds(0, CHUNK)], sem)
    c3 = pltpu.async_copy(et_hbm.at[pl.ds(base, CHUNK)],
                          tv.at[pl.ds(0, CHUNK)], sem)
    c4 = pltpu.async_copy(head_hbm, hv, sem)
    c5 = pltpu.async_copy(tail_hbm, tlv, sem)
    with jax.named_scope("dma_in"):
        for c in (z1, z2, z3, c1, c2, c3, c4, c5):
            c.wait()

    lanes = lax.iota(jnp.int32, 16)
    with jax.named_scope("build"):
        vmask = lanes < 8
        padv = jnp.full((16,), PADNODE, jnp.int32)
        sv[pl.ds(CHUNK - 8, 16)] = jnp.where(vmask, sv[pl.ds(CHUNK - 8, 16)],
                                             PADNODE)
        dv[pl.ds(CHUNK - 8, 16)] = jnp.where(vmask, dv[pl.ds(CHUNK - 8, 16)],
                                             PADNODE)
        for p in range(CHUNK + 8, CPAD, 16):
            sv[pl.ds(p, 16)] = padv
            dv[pl.ds(p, 16)] = padv

        lane0 = lanes == 0
        hv0 = hv[pl.ds(0, 16)]
        hv1 = hv[pl.ds(16, 16)]
        tv0 = tlv[pl.ds(0, 16)]
        tv1 = tlv[pl.ds(16, 16)]
        for b in range(B):
            hid = hv0[b] if b < 16 else hv1[b - 16]
            tid = tv0[b] if b < 16 else tv1[b - 16]
            bit = jnp.full((16,), _BITS[b], jnp.int32)
            hidx = jnp.full((16,), hid, jnp.int32)
            tidx = jnp.full((16,), tid, jnp.int32)
            hold = plsc.load_gather(htbl, [hidx], mask=lane0)
            plsc.store_scatter(htbl, [hidx], hold | bit, mask=lane0)
            told = plsc.load_gather(ttbl, [tidx], mask=lane0)
            plsc.store_scatter(ttbl, [tidx], told | bit, mask=lane0)
            ahold = plsc.load_gather(atbl, [hidx], mask=lane0)
            plsc.store_scatter(atbl, [hidx], ahold | bit, mask=lane0)
            atold = plsc.load_gather(atbl, [tidx], mask=lane0)
            plsc.store_scatter(atbl, [tidx], atold | bit, mask=lane0)

    def _scan(j, cnt):
        base4 = j * 64
        ss = [sv[pl.ds(base4 + u * 16, 16)] for u in range(4)]
        dd = [dv[pl.ds(base4 + u * 16, 16)] for u in range(4)]
        aas = [plsc.load_gather(atbl, [s]) for s in ss]
        aad = [plsc.load_gather(atbl, [d]) for d in dd]
        msks = [(aas[u] | aad[u]) != 0 for u in range(4)]
        for u in range(4):
            plsc.store_compressed(queue.at[pl.ds(cnt, 16)],
                                  base4 + u * 16 + lanes, mask=msks[u])
            cnt = cnt + plsc.all_reduce_population_count(msks[u])[0]
        return cnt

    with jax.named_scope("scan"):
        cnt = lax.fori_loop(0, NV // 4, _scan, jnp.int32(0))

    with jax.named_scope("czwait"):
        cz.wait()
    onesf = jnp.ones((16,), jnp.float32)
    queue[pl.ds(cnt, 16)] = jnp.full((16,), CHUNK, jnp.int32)

    def _proc(k, c):
        e16 = queue[pl.ds(k * 16, 16)]
        s16 = plsc.load_gather(sv, [e16])
        d16 = plsc.load_gather(dv, [e16])
        t16 = plsc.load_gather(tv, [e16])
        hs = plsc.load_gather(htbl, [s16])
        hd = plsc.load_gather(htbl, [d16])
        ts = plsc.load_gather(ttbl, [s16])
        td = plsc.load_gather(ttbl, [d16])
        m5 = hs & td
        m6 = hd & ts
        modes = [hd & ~m6, hs & ~m5, td & ~m5, ts & ~m6, m5, m6]
        for j in range(16):
            tvec = jnp.full((16,), t16[j], jnp.int32)
            mj = [jnp.full((16,), modes[i][j], jnp.int32) for i in range(L)]
            for half in range(2):
                bidx = lanes + (16 * half)
                for i in range(L):
                    bits = jnp.right_shift(mj[i], bidx) & 1
                    msk = bits == 1
                    col = bidx + (i * B)
                    plsc.addupdate_scatter(cbuf, [tvec, col], onesf, mask=msk)
        return c

    with jax.named_scope("proc"):
        lax.fori_loop(0, (cnt + 15) // 16, _proc, 0)
    with jax.named_scope("out"):
        pltpu.sync_copy(cbuf, out_hbm.at[wid])


def _tail_kernel(cp_ref, lab_ref, rv_ref, W1_ref, b1_ref, W2_ref, b2_ref,
                 reldW_ref, reldb_ref, concW_ref, concb_ref, fcW_ref, fcb_ref,
                 out_ref, *, n_rels, b_rows, link_mode):
    CT = jnp.sum(cp_ref[...], axis=0)
    V = (jnp.dot(rv_ref[...], W1_ref[...],
                 preferred_element_type=jnp.float32) + b1_ref[...])
    V = (jnp.dot(V, W2_ref[...],
                 preferred_element_type=jnp.float32) + b2_ref[...])
    S = lax.dot_general(CT, V, (((0,), (0,)), ((), ())),
                        preferred_element_type=jnp.float32)
    n = jnp.sum(CT, axis=0)[:, None]
    acc = jnp.zeros((b_rows, V.shape[1]), jnp.float32)
    for m in range(link_mode):
        Sm = S[m * b_rows:(m + 1) * b_rows, :]
        nm = n[m * b_rows:(m + 1) * b_rows, :]
        Tm = (jnp.dot(Sm, reldW_ref[m], preferred_element_type=jnp.float32)
              + nm * reldb_ref[m, :][None, :])
        acc = acc + Tm / (nm + 1e-30)
    rel_neighbor = acc / float(link_mode)

    lab = lab_ref[:, 0]
    loh = (lab[:, None] == jax.lax.broadcasted_iota(
        jnp.int32, (b_rows, n_rels), 1)).astype(jnp.float32)
    rel_embeds = jnp.dot(loh, V, preferred_element_type=jnp.float32)

    hcat = jnp.concatenate([rel_neighbor, rel_embeds], axis=1)
    hh = (jnp.dot(hcat, concW_ref[...], preferred_element_type=jnp.float32)
          + concb_ref[...])
    hh = jnp.maximum(hh, 0.0)
    nrm = jnp.sqrt(jnp.sum(hh * hh, axis=1, keepdims=True))
    g = hh / jnp.maximum(nrm, 1e-12)
    out_ref[...] = (jnp.dot(g, fcW_ref[...],
                            preferred_element_type=jnp.float32) + fcb_ref[...])


def kernel(src, dst, edge_type, head_ids, tail_ids, rel_labels, rel_vectors,
           W1, b1, W2, b2, reld_W, reld_b, conc_W, conc_b, fc_W, fc_b):
    DV = rel_vectors.shape[1]
    D = W1.shape[1]

    mesh = plsc.VectorSubcoreMesh(core_axis_name="c", subcore_axis_name="s")

    hist = pl.kernel(
        _sc_hist,
        mesh=mesh,
        compiler_params=pltpu.CompilerParams(needs_layout_passes=False),
        out_type=jax.ShapeDtypeStruct((NWORKERS, R, LB), jnp.float32),
        scratch_types=[
            pltpu.VMEM((EBUF,), jnp.int32),
            pltpu.VMEM((EBUF,), jnp.int32),
            pltpu.VMEM((EBUF,), jnp.int32),
            pltpu.VMEM((NPAD,), jnp.int32),
            pltpu.VMEM((NPAD,), jnp.int32),
            pltpu.VMEM((NPAD,), jnp.int32),
            pltpu.VMEM((EBUF,), jnp.int32),
            pltpu.VMEM((R, LB), jnp.float32),
            pltpu.VMEM((B,), jnp.int32),
            pltpu.VMEM((B,), jnp.int32),
            pltpu.SemaphoreType.DMA,
            pltpu.SemaphoreType.DMA,
        ],
    )
    zt = jnp.zeros((NPAD,), jnp.int32)
    zc = jnp.zeros((R, LB), jnp.float32)
    cparts = hist(src, dst, edge_type, head_ids, tail_ids, zt, zc)

    def blk(shape):
        return pl.BlockSpec(shape, lambda *, _s=shape: tuple(0 for _ in _s))

    out = pl.pallas_call(
        functools.partial(_tail_kernel, n_rels=R, b_rows=B, link_mode=L),
        in_specs=[
            blk((NWORKERS, R, LB)),
            blk((B, 1)),
            blk((R, DV)),
            blk((DV, D)), blk((1, D)),
            blk((D, D)), blk((1, D)),
            blk((L, D, D)), blk((L, D)),
            blk((2 * D, D)), blk((1, D)),
            blk((D, 1)), blk((1, 1)),
        ],
        out_specs=blk((B, 1)),
        out_shape=jax.ShapeDtypeStruct((B, 1), jnp.float32),
    )(cparts, rel_labels.reshape(B, 1),
      rel_vectors, W1, b1.reshape(1, D), W2, b2.reshape(1, D),
      reld_W, reld_b, conc_W, conc_b.reshape(1, D),
      fc_W, fc_b.reshape(1, 1))
    return out


# speedup vs baseline: 12.0159x; 1.0436x over previous
import functools
import numpy as np
import jax
import jax.numpy as jnp
from jax import lax
from jax.experimental import pallas as pl
from jax.experimental.pallas import tpu as pltpu
from jax.experimental.pallas import tpu_sc as plsc

E_EDGES = 160000
NWORKERS = 32
CHUNK = E_EDGES // NWORKERS
CPAD = CHUNK + 56
NV = CPAD // 16
EBUF = CPAD + 32
N_NODES = 10000
NPAD = N_NODES + 32
PADNODE = N_NODES
B = 32
R = 200
L = 6
LB = L * B

def _sc_hist(src_hbm, dst_hbm, et_hbm, head_hbm, tail_hbm, zt_hbm, zc_hbm,
             out_hbm, sv, dv, tv, htbl, ttbl, atbl, queue, cbuf, hv, tlv,
             sem, semb, csem):
    wid = lax.axis_index("c") * 16 + lax.axis_index("s")
    base = wid * CHUNK
    cz = pltpu.async_copy(zc_hbm, cbuf, csem)
    c3 = pltpu.async_copy(et_hbm.at[pl.ds(base, CHUNK)],
                          tv.at[pl.ds(0, CHUNK)], csem)
    z1 = pltpu.async_copy(zt_hbm, htbl, sem)
    z2 = pltpu.async_copy(zt_hbm, ttbl, sem)
    z3 = pltpu.async_copy(zt_hbm, atbl, sem)
    c1 = pltpu.async_copy(src_hbm.at[pl.ds(base, CHUNK)],
                          sv.at[pl.ds(0, CHUNK)], semb)
    c2 = pltpu.async_copy(dst_hbm.at[pl.ds(base, CHUNK)],
                          dv.at[pl.ds(0, CHUNK)], semb)
    c4 = pltpu.async_copy(head_hbm, hv, sem)
    c5 = pltpu.async_copy(tail_hbm, tlv, sem)
    with jax.named_scope("dma_tbl"):
        for c in (z1, z2, z3, c4, c5):
            c.wait()

    lanes = lax.iota(jnp.int32, 16)
    with jax.named_scope("build"):
        bits_lo = jnp.left_shift(jnp.full((16,), 1, jnp.int32), lanes)
        bits_hi = jnp.left_shift(jnp.full((16,), 1, jnp.int32), lanes + 16)
        onesi = jnp.ones((16,), jnp.int32)
        hv0 = hv[pl.ds(0, 16)]
        hv1 = hv[pl.ds(16, 16)]
        tv0 = tlv[pl.ds(0, 16)]
        tv1 = tlv[pl.ds(16, 16)]
        plsc.addupdate_scatter(htbl, [hv0], bits_lo)
        plsc.addupdate_scatter(htbl, [hv1], bits_hi)
        plsc.addupdate_scatter(ttbl, [tv0], bits_lo)
        plsc.addupdate_scatter(ttbl, [tv1], bits_hi)
        plsc.store_scatter(atbl, [hv0], onesi)
        plsc.store_scatter(atbl, [hv1], onesi)
        plsc.store_scatter(atbl, [tv0], onesi)
        plsc.store_scatter(atbl, [tv1], onesi)

    with jax.named_scope("dma_edges"):
        for c in (c1, c2):
            c.wait()
    with jax.named_scope("pad"):
        vmask = lanes < 8
        padv = jnp.full((16,), PADNODE, jnp.int32)
        sv[pl.ds(CHUNK - 8, 16)] = jnp.where(vmask, sv[pl.ds(CHUNK - 8, 16)],
                                             PADNODE)
        dv[pl.ds(CHUNK - 8, 16)] = jnp.where(vmask, dv[pl.ds(CHUNK - 8, 16)],
                                             PADNODE)
        for p in range(CHUNK + 8, CPAD, 16):
            sv[pl.ds(p, 16)] = padv
            dv[pl.ds(p, 16)] = padv

    def _scan(j, cnt):
        base4 = j * 64
        ss = [sv[pl.ds(base4 + u * 16, 16)] for u in range(4)]
        dd = [dv[pl.ds(base4 + u * 16, 16)] for u in range(4)]
        aas = [plsc.load_gather(atbl, [s]) for s in ss]
        aad = [plsc.load_gather(atbl, [d]) for d in dd]
        msks = [(aas[u] | aad[u]) != 0 for u in range(4)]
        for u in range(4):
            plsc.store_compressed(queue.at[pl.ds(cnt, 16)],
                                  base4 + u * 16 + lanes, mask=msks[u])
            cnt = cnt + plsc.all_reduce_population_count(msks[u])[0]
        return cnt

    with jax.named_scope("scan"):
        cnt = lax.fori_loop(0, NV // 4, _scan, jnp.int32(0))

    with jax.named_scope("czwait"):
        cz.wait()
        c3.wait()
    onesf = jnp.ones((16,), jnp.float32)
    queue[pl.ds(cnt, 16)] = jnp.full((16,), CHUNK, jnp.int32)

    def _proc(k, c):
        e16 = queue[pl.ds(k * 16, 16)]
        s16 = plsc.load_gather(sv, [e16])
        d16 = plsc.load_gather(dv, [e16])
        t16 = plsc.load_gather(tv, [e16])
        hs = plsc.load_gather(htbl, [s16])
        hd = plsc.load_gather(htbl, [d16])
        ts = plsc.load_gather(ttbl, [s16])
        td = plsc.load_gather(ttbl, [d16])
        m5 = hs & td
        m6 = hd & ts
        modes = [hd & ~m6, hs & ~m5, td & ~m5, ts & ~m6, m5, m6]
        for j in range(16):
            tvec = jnp.full((16,), t16[j], jnp.int32)
            mj = [jnp.full((16,), modes[i][j], jnp.int32) for i in range(L)]
            for half in range(2):
                bidx = lanes + (16 * half)
                for i in range(L):
                    bits = jnp.right_shift(mj[i], bidx) & 1
                    msk = bits == 1
                    col = bidx + (i * B)
                    plsc.addupdate_scatter(cbuf, [tvec, col], onesf, mask=msk)
        return c

    with jax.named_scope("proc"):
        lax.fori_loop(0, (cnt + 15) // 16, _proc, 0)
    with jax.named_scope("out"):
        pltpu.sync_copy(cbuf, out_hbm.at[wid])


def _tail_kernel(cp_ref, lab_ref, rv_ref, W1_ref, b1_ref, W2_ref, b2_ref,
                 reldW_ref, reldb_ref, concW_ref, concb_ref, fcW_ref, fcb_ref,
                 out_ref, *, n_rels, b_rows, link_mode):
    CT = jnp.sum(cp_ref[...], axis=0)
    V = (jnp.dot(rv_ref[...], W1_ref[...],
                 preferred_element_type=jnp.float32) + b1_ref[...])
    V = (jnp.dot(V, W2_ref[...],
                 preferred_element_type=jnp.float32) + b2_ref[...])
    S = lax.dot_general(CT, V, (((0,), (0,)), ((), ())),
                        preferred_element_type=jnp.float32)
    n = jnp.sum(CT, axis=0)[:, None]
    acc = jnp.zeros((b_rows, V.shape[1]), jnp.float32)
    for m in range(link_mode):
        Sm = S[m * b_rows:(m + 1) * b_rows, :]
        nm = n[m * b_rows:(m + 1) * b_rows, :]
        Tm = (jnp.dot(Sm, reldW_ref[m], preferred_element_type=jnp.float32)
              + nm * reldb_ref[m, :][None, :])
        acc = acc + Tm / (nm + 1e-30)
    rel_neighbor = acc / float(link_mode)

    lab = lab_ref[:, 0]
    loh = (lab[:, None] == jax.lax.broadcasted_iota(
        jnp.int32, (b_rows, n_rels), 1)).astype(jnp.float32)
    rel_embeds = jnp.dot(loh, V, preferred_element_type=jnp.float32)

    hcat = jnp.concatenate([rel_neighbor, rel_embeds], axis=1)
    hh = (jnp.dot(hcat, concW_ref[...], preferred_element_type=jnp.float32)
          + concb_ref[...])
    hh = jnp.maximum(hh, 0.0)
    nrm = jnp.sqrt(jnp.sum(hh * hh, axis=1, keepdims=True))
    g = hh / jnp.maximum(nrm, 1e-12)
    out_ref[...] = (jnp.dot(g, fcW_ref[...],
                            preferred_element_type=jnp.float32) + fcb_ref[...])


def kernel(src, dst, edge_type, head_ids, tail_ids, rel_labels, rel_vectors,
           W1, b1, W2, b2, reld_W, reld_b, conc_W, conc_b, fc_W, fc_b):
    DV = rel_vectors.shape[1]
    D = W1.shape[1]

    mesh = plsc.VectorSubcoreMesh(core_axis_name="c", subcore_axis_name="s")

    hist = pl.kernel(
        _sc_hist,
        mesh=mesh,
        compiler_params=pltpu.CompilerParams(needs_layout_passes=False),
        out_type=jax.ShapeDtypeStruct((NWORKERS, R, LB), jnp.float32),
        scratch_types=[
            pltpu.VMEM((EBUF,), jnp.int32),
            pltpu.VMEM((EBUF,), jnp.int32),
            pltpu.VMEM((EBUF,), jnp.int32),
            pltpu.VMEM((NPAD,), jnp.int32),
            pltpu.VMEM((NPAD,), jnp.int32),
            pltpu.VMEM((NPAD,), jnp.int32),
            pltpu.VMEM((EBUF,), jnp.int32),
            pltpu.VMEM((R, LB), jnp.float32),
            pltpu.VMEM((B,), jnp.int32),
            pltpu.VMEM((B,), jnp.int32),
            pltpu.SemaphoreType.DMA,
            pltpu.SemaphoreType.DMA,
            pltpu.SemaphoreType.DMA,
        ],
    )
    zt = jnp.zeros((NPAD,), jnp.int32)
    zc = jnp.zeros((R, LB), jnp.float32)
    cparts = hist(src, dst, edge_type, head_ids, tail_ids, zt, zc)

    def blk(shape):
        return pl.BlockSpec(shape, lambda *, _s=shape: tuple(0 for _ in _s))

    out = pl.pallas_call(
        functools.partial(_tail_kernel, n_rels=R, b_rows=B, link_mode=L),
        in_specs=[
            blk((NWORKERS, R, LB)),
            blk((B, 1)),
            blk((R, DV)),
            blk((DV, D)), blk((1, D)),
            blk((D, D)), blk((1, D)),
            blk((L, D, D)), blk((L, D)),
            blk((2 * D, D)), blk((1, D)),
            blk((D, 1)), blk((1, 1)),
        ],
        out_specs=blk((B, 1)),
        out_shape=jax.ShapeDtypeStruct((B, 1), jnp.float32),
    )(cparts, rel_labels.reshape(B, 1),
      rel_vectors, W1, b1.reshape(1, D), W2, b2.reshape(1, D),
      reld_W, reld_b, conc_W, conc_b.reshape(1, D),
      fc_W, fc_b.reshape(1, 1))
    return out


# speedup vs baseline: 12.1410x; 1.0104x over previous
import functools
import numpy as np
import jax
import jax.numpy as jnp
from jax import lax
from jax.experimental import pallas as pl
from jax.experimental.pallas import tpu as pltpu
from jax.experimental.pallas import tpu_sc as plsc

E_EDGES = 160000
NWORKERS = 32
CHUNK = E_EDGES // NWORKERS
CPAD = CHUNK + 56
NV = CPAD // 16
EBUF = CPAD + 32
N_NODES = 10000
NPAD = N_NODES + 32
PADNODE = N_NODES
B = 32
R = 200
L = 6
LB = L * B

def _sc_hist(src_hbm, dst_hbm, et_hbm, head_hbm, tail_hbm, zt_hbm, zc_hbm,
             out_hbm, sv, dv, tv, htbl, ttbl, atbl, queue, cbuf, hv, tlv,
             sem, semb, csem):
    wid = lax.axis_index("c") * 16 + lax.axis_index("s")
    base = wid * CHUNK
    cz = pltpu.async_copy(zc_hbm, cbuf, csem)
    c3 = pltpu.async_copy(et_hbm.at[pl.ds(base, CHUNK)],
                          tv.at[pl.ds(0, CHUNK)], csem)
    z1 = pltpu.async_copy(zt_hbm, htbl, sem)
    z2 = pltpu.async_copy(zt_hbm, ttbl, sem)
    z3 = pltpu.async_copy(zt_hbm, atbl, sem)
    c1 = pltpu.async_copy(src_hbm.at[pl.ds(base, CHUNK)],
                          sv.at[pl.ds(0, CHUNK)], semb)
    c2 = pltpu.async_copy(dst_hbm.at[pl.ds(base, CHUNK)],
                          dv.at[pl.ds(0, CHUNK)], semb)
    c4 = pltpu.async_copy(head_hbm, hv, sem)
    c5 = pltpu.async_copy(tail_hbm, tlv, sem)
    with jax.named_scope("dma_tbl"):
        for c in (z1, z2, z3, c4, c5):
            c.wait()

    lanes = lax.iota(jnp.int32, 16)
    with jax.named_scope("build"):
        bits_lo = jnp.left_shift(jnp.full((16,), 1, jnp.int32), lanes)
        bits_hi = jnp.left_shift(jnp.full((16,), 1, jnp.int32), lanes + 16)
        onesi = jnp.ones((16,), jnp.int32)
        hv0 = hv[pl.ds(0, 16)]
        hv1 = hv[pl.ds(16, 16)]
        tv0 = tlv[pl.ds(0, 16)]
        tv1 = tlv[pl.ds(16, 16)]
        plsc.addupdate_scatter(htbl, [hv0], bits_lo)
        plsc.addupdate_scatter(htbl, [hv1], bits_hi)
        plsc.addupdate_scatter(ttbl, [tv0], bits_lo)
        plsc.addupdate_scatter(ttbl, [tv1], bits_hi)
        plsc.store_scatter(atbl, [hv0], onesi)
        plsc.store_scatter(atbl, [hv1], onesi)
        plsc.store_scatter(atbl, [tv0], onesi)
        plsc.store_scatter(atbl, [tv1], onesi)

    with jax.named_scope("dma_edges"):
        for c in (c1, c2):
            c.wait()
    with jax.named_scope("pad"):
        vmask = lanes < 8
        padv = jnp.full((16,), PADNODE, jnp.int32)
        sv[pl.ds(CHUNK - 8, 16)] = jnp.where(vmask, sv[pl.ds(CHUNK - 8, 16)],
                                             PADNODE)
        dv[pl.ds(CHUNK - 8, 16)] = jnp.where(vmask, dv[pl.ds(CHUNK - 8, 16)],
                                             PADNODE)
        for p in range(CHUNK + 8, CPAD, 16):
            sv[pl.ds(p, 16)] = padv
            dv[pl.ds(p, 16)] = padv

    def _scan(j, cnt):
        base4 = j * 64
        ss = [sv[pl.ds(base4 + u * 16, 16)] for u in range(4)]
        dd = [dv[pl.ds(base4 + u * 16, 16)] for u in range(4)]
        aas = [plsc.load_gather(atbl, [s]) for s in ss]
        aad = [plsc.load_gather(atbl, [d]) for d in dd]
        msks = [(aas[u] | aad[u]) != 0 for u in range(4)]
        for u in range(4):
            plsc.store_compressed(queue.at[pl.ds(cnt, 16)],
                                  base4 + u * 16 + lanes, mask=msks[u])
            cnt = cnt + plsc.all_reduce_population_count(msks[u])[0]
        return cnt

    with jax.named_scope("scan"):
        cnt = lax.fori_loop(0, NV // 4, _scan, jnp.int32(0))

    with jax.named_scope("czwait"):
        cz.wait()
        c3.wait()
    onesf = jnp.ones((16,), jnp.float32)
    queue[pl.ds(cnt, 16)] = jnp.full((16,), CHUNK, jnp.int32)

    def _proc(k, c):
        e16 = queue[pl.ds(k * 16, 16)]
        s16 = plsc.load_gather(sv, [e16])
        d16 = plsc.load_gather(dv, [e16])
        t16 = plsc.load_gather(tv, [e16])
        hs = plsc.load_gather(htbl, [s16])
        hd = plsc.load_gather(htbl, [d16])
        ts = plsc.load_gather(ttbl, [s16])
        td = plsc.load_gather(ttbl, [d16])
        m5 = hs & td
        m6 = hd & ts
        modes = [hd & ~m6, hs & ~m5, td & ~m5, ts & ~m6, m5, m6]
        u0 = hs | hd | ts | td

        def _peel(u):
            nz = u != 0
            low = u & (0 - u)
            fexp = plsc.bitcast(low.astype(jnp.float32), jnp.int32)
            b = (jnp.right_shift(fexp, 23) & 255) - 127
            b = jnp.where(nz, b, 0)
            for i in range(L):
                mi = (modes[i] & low) != 0
                plsc.addupdate_scatter(cbuf, [t16, b + (i * B)], onesf,
                                       mask=mi)
            return u & (u - 1)

        def _any(u):
            return plsc.all_reduce_population_count(u != 0)[0] > 0

        u1 = _peel(u0)

        @pl.when(_any(u1))
        def _():
            u2 = _peel(u1)

            @pl.when(_any(u2))
            def _():
                u = u2
                for _ in range(B - 2):
                    u = _peel(u)
        return c

    with jax.named_scope("proc"):
        lax.fori_loop(0, (cnt + 15) // 16, _proc, 0)
    with jax.named_scope("out"):
        pltpu.sync_copy(cbuf, out_hbm.at[wid])


def _tail_kernel(cp_ref, lab_ref, rv_ref, W1_ref, b1_ref, W2_ref, b2_ref,
                 reldW_ref, reldb_ref, concW_ref, concb_ref, fcW_ref, fcb_ref,
                 out_ref, *, n_rels, b_rows, link_mode):
    CT = jnp.sum(cp_ref[...], axis=0)
    V = (jnp.dot(rv_ref[...], W1_ref[...],
                 preferred_element_type=jnp.float32) + b1_ref[...])
    V = (jnp.dot(V, W2_ref[...],
                 preferred_element_type=jnp.float32) + b2_ref[...])
    S = lax.dot_general(CT, V, (((0,), (0,)), ((), ())),
                        preferred_element_type=jnp.float32)
    n = jnp.sum(CT, axis=0)[:, None]
    acc = jnp.zeros((b_rows, V.shape[1]), jnp.float32)
    for m in range(link_mode):
        Sm = S[m * b_rows:(m + 1) * b_rows, :]
        nm = n[m * b_rows:(m + 1) * b_rows, :]
        Tm = (jnp.dot(Sm, reldW_ref[m], preferred_element_type=jnp.float32)
              + nm * reldb_ref[m, :][None, :])
        acc = acc + Tm / (nm + 1e-30)
    rel_neighbor = acc / float(link_mode)

    lab = lab_ref[:, 0]
    loh = (lab[:, None] == jax.lax.broadcasted_iota(
        jnp.int32, (b_rows, n_rels), 1)).astype(jnp.float32)
    rel_embeds = jnp.dot(loh, V, preferred_element_type=jnp.float32)

    hcat = jnp.concatenate([rel_neighbor, rel_embeds], axis=1)
    hh = (jnp.dot(hcat, concW_ref[...], preferred_element_type=jnp.float32)
          + concb_ref[...])
    hh = jnp.maximum(hh, 0.0)
    nrm = jnp.sqrt(jnp.sum(hh * hh, axis=1, keepdims=True))
    g = hh / jnp.maximum(nrm, 1e-12)
    out_ref[...] = (jnp.dot(g, fcW_ref[...],
                            preferred_element_type=jnp.float32) + fcb_ref[...])


def kernel(src, dst, edge_type, head_ids, tail_ids, rel_labels, rel_vectors,
           W1, b1, W2, b2, reld_W, reld_b, conc_W, conc_b, fc_W, fc_b):
    DV = rel_vectors.shape[1]
    D = W1.shape[1]

    mesh = plsc.VectorSubcoreMesh(core_axis_name="c", subcore_axis_name="s")

    hist = pl.kernel(
        _sc_hist,
        mesh=mesh,
        compiler_params=pltpu.CompilerParams(needs_layout_passes=False),
        out_type=jax.ShapeDtypeStruct((NWORKERS, R, LB), jnp.float32),
        scratch_types=[
            pltpu.VMEM((EBUF,), jnp.int32),
            pltpu.VMEM((EBUF,), jnp.int32),
            pltpu.VMEM((EBUF,), jnp.int32),
            pltpu.VMEM((NPAD,), jnp.int32),
            pltpu.VMEM((NPAD,), jnp.int32),
            pltpu.VMEM((NPAD,), jnp.int32),
            pltpu.VMEM((EBUF,), jnp.int32),
            pltpu.VMEM((R, LB), jnp.float32),
            pltpu.VMEM((B,), jnp.int32),
            pltpu.VMEM((B,), jnp.int32),
            pltpu.SemaphoreType.DMA,
            pltpu.SemaphoreType.DMA,
            pltpu.SemaphoreType.DMA,
        ],
    )
    zt = jnp.zeros((NPAD,), jnp.int32)
    zc = jnp.zeros((R, LB), jnp.float32)
    cparts = hist(src, dst, edge_type, head_ids, tail_ids, zt, zc)

    def blk(shape):
        return pl.BlockSpec(shape, lambda *, _s=shape: tuple(0 for _ in _s))

    out = pl.pallas_call(
        functools.partial(_tail_kernel, n_rels=R, b_rows=B, link_mode=L),
        in_specs=[
            blk((NWORKERS, R, LB)),
            blk((B, 1)),
            blk((R, DV)),
            blk((DV, D)), blk((1, D)),
            blk((D, D)), blk((1, D)),
            blk((L, D, D)), blk((L, D)),
            blk((2 * D, D)), blk((1, D)),
            blk((D, 1)), blk((1, 1)),
        ],
        out_specs=blk((B, 1)),
        out_shape=jax.ShapeDtypeStruct((B, 1), jnp.float32),
    )(cparts, rel_labels.reshape(B, 1),
      rel_vectors, W1, b1.reshape(1, D), W2, b2.reshape(1, D),
      reld_W, reld_b, conc_W, conc_b.reshape(1, D),
      fc_W, fc_b.reshape(1, 1))
    return out
